# baseline (speedup 1.0000x reference)
import jax
import jax.numpy as jnp
from jax import lax
from jax.experimental import pallas as pl
from jax.experimental.pallas import tpu as pltpu
from jax.experimental.pallas import tpu_sc as plsc

N_NODES = 10000
N_PAD = 10240
N_EDGES = 320000
D = 128

NC = 2
NS = 16
NW = NC * NS
CHUNK = 128
NCHUNK = 78
EPW = NCHUNK * CHUNK
E_MAIN = EPW * NW
TAIL = (N_EDGES - E_MAIN) // NW
ROWS_PER_SUB = N_PAD // NS

_SC_PARAMS = pltpu.CompilerParams(needs_layout_passes=False)


def _make_agg():
    mesh = plsc.VectorSubcoreMesh(core_axis_name="c", subcore_axis_name="s")
    out_type = [jax.ShapeDtypeStruct((NC * N_PAD, D), jnp.float32)]
    scratch = [
        pltpu.VMEM((NCHUNK, CHUNK), jnp.int32),
        pltpu.VMEM((CHUNK,), jnp.int32),
        pltpu.VMEM((CHUNK,), jnp.int32),
        pltpu.VMEM((CHUNK,), jnp.int32),
        pltpu.VMEM((CHUNK,), jnp.int32),
        pltpu.VMEM((TAIL,), jnp.int32),
        pltpu.VMEM((TAIL,), jnp.int32),
        pltpu.VMEM((CHUNK, D), jnp.float32),
        pltpu.VMEM((CHUNK, D), jnp.float32),
        pltpu.VMEM_SHARED((N_PAD, D), jnp.float32),
        pltpu.SemaphoreType.DMA,
        pltpu.SemaphoreType.DMA,
        pltpu.SemaphoreType.DMA,
        pltpu.SemaphoreType.DMA,
    ]
    def body(pk_hbm, x_hbm, srcall_hbm, dstall_hbm, *refs):
        (acc_out, pk_v, srcb0, srcb1, dstb0, dstb1, tsrc_v,
         tdst_v, rows0, rows1, acc, semg0, semg1, sems0, sems1) = refs
        c = lax.axis_index("c")
        s = lax.axis_index("s")
        w = s * NC + c
        pltpu.sync_copy(pk_hbm.at[w], pk_v)
        toff = pl.multiple_of(E_MAIN + w * TAIL, TAIL)
        pltpu.sync_copy(srcall_hbm.at[pl.ds(toff, TAIL)], tsrc_v)
        pltpu.sync_copy(dstall_hbm.at[pl.ds(toff, TAIL)], tdst_v)

        def unpack_src(i, sb):
            for j in range(CHUNK // 16):
                v = pk_v[i, pl.ds(j * 16, 16)]
                sb[pl.ds(j * 16, 16)] = lax.shift_right_logical(v, 16)

        def unpack_dst(i, db):
            for j in range(CHUNK // 16):
                v = pk_v[i, pl.ds(j * 16, 16)]
                db[pl.ds(j * 16, 16)] = jnp.bitwise_and(v, 0xFFFF)

        z16 = jnp.zeros((16,), jnp.float32)

        def zrow_store(r, carry):
            for l in range(D // 16):
                rows0[r, pl.ds(l * 16, 16)] = z16
            return carry

        lax.fori_loop(0, CHUNK, zrow_store, 0)
        row0 = pl.multiple_of(s * ROWS_PER_SUB, ROWS_PER_SUB)
        zcopies = [
            pltpu.async_copy(
                rows0, acc.at[pl.ds(row0 + k * CHUNK, CHUNK)], semg0)
            for k in range(ROWS_PER_SUB // CHUNK)
        ]
        for zc in zcopies:
            zc.wait()
        plsc.subcore_barrier()

        unpack_src(0, srcb0)
        pltpu.async_copy(x_hbm.at[srcb0], rows0, semg0)
        unpack_src(1, srcb1)
        pltpu.async_copy(x_hbm.at[srcb1], rows1, semg1)
        bufs = ((rows0, semg0, srcb0), (rows1, semg1, srcb1))

        def outer(i0, carry):
            for b, (rv, sm, sb) in enumerate(bufs):
                i = i0 + b
                pltpu.make_async_copy(x_hbm.at[pl.ds(0, CHUNK)], rv, sm).wait()
                unpack_dst(i, dstb0)
                pltpu.sync_copy(rv, acc.at[dstb0], add=True)
                @pl.when(i + 2 < NCHUNK)
                def _():
                    unpack_src(i + 2, sb)
                    pltpu.async_copy(x_hbm.at[sb], rv, sm)
            return carry

        lax.fori_loop(0, NCHUNK // 2, lambda k, cr: outer(k * 2, cr), 0)
        pltpu.async_copy(x_hbm.at[tsrc_v], rows0.at[pl.ds(0, TAIL)], semg0)
        pltpu.make_async_copy(
            x_hbm.at[pl.ds(0, TAIL)], rows0.at[pl.ds(0, TAIL)], semg0).wait()
        pltpu.sync_copy(rows0.at[pl.ds(0, TAIL)], acc.at[tdst_v], add=True)
        plsc.subcore_barrier()
        ocopies = [
            pltpu.async_copy(
                acc.at[pl.ds(row0 + k * CHUNK, CHUNK)],
                acc_out.at[pl.ds(c * N_PAD + row0 + k * CHUNK, CHUNK)],
                semg1)
            for k in range(ROWS_PER_SUB // CHUNK)
        ]
        for oc in ocopies:
            oc.wait()

    return pl.kernel(body, out_type=out_type, mesh=mesh,
                     scratch_types=scratch, compiler_params=_SC_PARAMS)


_agg = _make_agg()


def _make_cnt():
    mesh = plsc.VectorSubcoreMesh(core_axis_name="c", subcore_axis_name="s")
    out_type = [jax.ShapeDtypeStruct((NW * N_PAD,), jnp.float32),
                jax.ShapeDtypeStruct((NW, NCHUNK, CHUNK), jnp.int32)]
    scratch = [
        pltpu.VMEM((EPW,), jnp.int32),
        pltpu.VMEM((EPW,), jnp.int32),
        pltpu.VMEM((NCHUNK, CHUNK), jnp.int32),
        pltpu.VMEM((TAIL,), jnp.int32),
        pltpu.VMEM((N_PAD,), jnp.float32),
    ]

    def body(srcall_hbm, dstall_hbm, cnt_out, pk_out, src_v, dst_v,
             pk_v, tdst_v, cnt_v):
        c = lax.axis_index("c")
        s_ = lax.axis_index("s")
        w = s_ * NC + c
        off = pl.multiple_of(w * EPW, EPW)
        pltpu.sync_copy(srcall_hbm.at[pl.ds(off, EPW)], src_v)
        pltpu.sync_copy(dstall_hbm.at[pl.ds(off, EPW)], dst_v)
        toff = pl.multiple_of(E_MAIN + w * TAIL, TAIL)
        pltpu.sync_copy(dstall_hbm.at[pl.ds(toff, TAIL)], tdst_v)
        z16 = jnp.zeros((16,), jnp.float32)
        ones = jnp.ones((16,), jnp.float32)

        def zcnt_store(r, carry):
            cnt_v[pl.ds(r * 16, 16)] = z16
            return carry

        lax.fori_loop(0, N_PAD // 16, zcnt_store, 0)

        def step(i, carry):
            for j in range(CHUNK // 16):
                vs = src_v[pl.ds(i * CHUNK + j * 16, 16)]
                vd = dst_v[pl.ds(i * CHUNK + j * 16, 16)]
                pk_v[i, pl.ds(j * 16, 16)] = jnp.bitwise_or(
                    lax.shift_left(vs, 16), vd)
                plsc.addupdate_scatter(cnt_v, [vd], ones)
            return carry

        lax.fori_loop(0, NCHUNK, step, 0)
        plsc.addupdate_scatter(cnt_v, [tdst_v[...]], ones)
        pltpu.sync_copy(cnt_v, cnt_out.at[pl.ds(w * N_PAD, N_PAD)])
        pltpu.sync_copy(pk_v, pk_out.at[w])

    return pl.kernel(body, out_type=out_type, mesh=mesh,
                     scratch_types=scratch, compiler_params=_SC_PARAMS)


_cnt = _make_cnt()

BLK = 2048


def _mm_body(x_ref, w_ref, b_ref, o_ref):
    o_ref[...] = lax.dot_general(
        x_ref[...], w_ref[...], (((1,), (1,)), ((), ())),
        preferred_element_type=jnp.float32) + b_ref[...]


def _mm(x, w, b):
    return pl.pallas_call(
        _mm_body,
        grid=(N_PAD // BLK,),
        in_specs=[
            pl.BlockSpec((BLK, D), lambda i: (i, 0)),
            pl.BlockSpec((D, D), lambda i: (0, 0)),
            pl.BlockSpec((1, D), lambda i: (0, 0)),
        ],
        out_specs=pl.BlockSpec((BLK, D), lambda i: (i, 0)),
        out_shape=jax.ShapeDtypeStruct((N_NODES, D), jnp.float32),
    )(x, w, b)


def _combine_body(acc_ref, cnt_ref, xwr_ref, wl_ref, o_ref):
    cnt = jnp.sum(cnt_ref[...], axis=0)
    tot = jnp.sum(acc_ref[...], axis=0)
    mean = tot / jnp.maximum(cnt, 1.0)[:, None]
    h = lax.dot_general(mean, wl_ref[...], (((1,), (1,)), ((), ())),
                        preferred_element_type=jnp.float32)
    o_ref[...] = jnp.maximum(h + xwr_ref[...], 0.0)


def _combine(acc, cnt, xwr, wl):
    grid = (N_PAD // BLK,)
    return pl.pallas_call(
        _combine_body,
        grid=grid,
        in_specs=[
            pl.BlockSpec((NC, BLK, D), lambda i: (0, i, 0)),
            pl.BlockSpec((NW, BLK), lambda i: (0, i)),
            pl.BlockSpec((BLK, D), lambda i: (i, 0)),
            pl.BlockSpec((D, D), lambda i: (0, 0)),
        ],
        out_specs=pl.BlockSpec((BLK, D), lambda i: (i, 0)),
        out_shape=jax.ShapeDtypeStruct((N_NODES, D), jnp.float32),
    )(acc, cnt, xwr, wl)


@jax.jit
def kernel(x, edge_index, Wl1, bl1, Wr1, Wl2, bl2, Wr2):
    src = edge_index[0].astype(jnp.int32)
    dst = edge_index[1].astype(jnp.int32)
    bl1r = bl1.reshape(1, D)
    bl2r = bl2.reshape(1, D)

    cnt, pkm = _cnt(src, dst)
    cnt = cnt.reshape(NW, N_PAD)
    xwr1 = _mm(x, Wr1, bl1r)
    (acc1,) = _agg(pkm, x, src, dst)
    h = _combine(acc1.reshape(NC, N_PAD, D), cnt, xwr1, Wl1)
    xwr2 = _mm(h, Wr2, bl2r)
    (acc2,) = _agg(pkm, h, src, dst)
    return _combine(acc2.reshape(NC, N_PAD, D), cnt, xwr2, Wl2)


# speedup vs baseline: 1.0069x; 1.0069x over previous
import jax
import jax.numpy as jnp
from jax import lax
from jax.experimental import pallas as pl
from jax.experimental.pallas import tpu as pltpu
from jax.experimental.pallas import tpu_sc as plsc

N_NODES = 10000
N_PAD = 10240
N_EDGES = 320000
D = 128

NC = 2
NS = 16
NW = NC * NS
CHUNK = 128
NCHUNK = 78
EPW = NCHUNK * CHUNK
E_MAIN = EPW * NW
TAIL = (N_EDGES - E_MAIN) // NW
ROWS_PER_SUB = N_PAD // NS

_SC_PARAMS = pltpu.CompilerParams(needs_layout_passes=False)


def _make_agg():
    mesh = plsc.VectorSubcoreMesh(core_axis_name="c", subcore_axis_name="s")
    out_type = [jax.ShapeDtypeStruct((NC * N_PAD, D), jnp.float32)]
    scratch = [
        pltpu.VMEM((NCHUNK, CHUNK), jnp.int32),
        pltpu.VMEM((CHUNK,), jnp.int32),
        pltpu.VMEM((CHUNK,), jnp.int32),
        pltpu.VMEM((CHUNK,), jnp.int32),
        pltpu.VMEM((CHUNK,), jnp.int32),
        pltpu.VMEM((TAIL,), jnp.int32),
        pltpu.VMEM((TAIL,), jnp.int32),
        pltpu.VMEM((CHUNK, D), jnp.float32),
        pltpu.VMEM((CHUNK, D), jnp.float32),
        pltpu.VMEM_SHARED((N_PAD, D), jnp.float32),
        pltpu.SemaphoreType.DMA,
        pltpu.SemaphoreType.DMA,
        pltpu.SemaphoreType.DMA,
        pltpu.SemaphoreType.DMA,
    ]
    def body(pk_hbm, x_hbm, srcall_hbm, dstall_hbm, *refs):
        (acc_out, pk_v, srcb0, srcb1, dstb0, dstb1, tsrc_v,
         tdst_v, rows0, rows1, acc, semg0, semg1, sems0, sems1) = refs
        c = lax.axis_index("c")
        s = lax.axis_index("s")
        w = s * NC + c
        pltpu.sync_copy(pk_hbm.at[w], pk_v)
        toff = pl.multiple_of(E_MAIN + w * TAIL, TAIL)
        pltpu.sync_copy(srcall_hbm.at[pl.ds(toff, TAIL)], tsrc_v)
        pltpu.sync_copy(dstall_hbm.at[pl.ds(toff, TAIL)], tdst_v)

        def unpack_src(i, sb):
            for j in range(CHUNK // 16):
                v = pk_v[i, pl.ds(j * 16, 16)]
                sb[pl.ds(j * 16, 16)] = lax.shift_right_logical(v, 16)

        def unpack_dst(i, db):
            for j in range(CHUNK // 16):
                v = pk_v[i, pl.ds(j * 16, 16)]
                db[pl.ds(j * 16, 16)] = jnp.bitwise_and(v, 0xFFFF)

        z16 = jnp.zeros((16,), jnp.float32)

        def zrow_store(r, carry):
            for l in range(D // 16):
                rows0[r, pl.ds(l * 16, 16)] = z16
            return carry

        lax.fori_loop(0, CHUNK, zrow_store, 0)
        row0 = pl.multiple_of(s * ROWS_PER_SUB, ROWS_PER_SUB)
        zcopies = [
            pltpu.async_copy(
                rows0, acc.at[pl.ds(row0 + k * CHUNK, CHUNK)], semg0)
            for k in range(ROWS_PER_SUB // CHUNK)
        ]
        for zc in zcopies:
            zc.wait()
        plsc.subcore_barrier()

        unpack_src(0, srcb0)
        pltpu.async_copy(x_hbm.at[srcb0], rows0, semg0)
        unpack_src(1, srcb1)
        pltpu.async_copy(x_hbm.at[srcb1], rows1, semg1)
        bufs = ((rows0, semg0, srcb0), (rows1, semg1, srcb1))

        def outer(i0, carry):
            for b, (rv, sm, sb) in enumerate(bufs):
                i = i0 + b
                pltpu.make_async_copy(x_hbm.at[pl.ds(0, CHUNK)], rv, sm).wait()
                unpack_dst(i, dstb0)
                pltpu.sync_copy(rv, acc.at[dstb0], add=True)
                @pl.when(i + 2 < NCHUNK)
                def _():
                    unpack_src(i + 2, sb)
                    pltpu.async_copy(x_hbm.at[sb], rv, sm)
            return carry

        lax.fori_loop(0, NCHUNK // 2, lambda k, cr: outer(k * 2, cr), 0)
        pltpu.async_copy(x_hbm.at[tsrc_v], rows0.at[pl.ds(0, TAIL)], semg0)
        pltpu.make_async_copy(
            x_hbm.at[pl.ds(0, TAIL)], rows0.at[pl.ds(0, TAIL)], semg0).wait()
        pltpu.sync_copy(rows0.at[pl.ds(0, TAIL)], acc.at[tdst_v], add=True)
        plsc.subcore_barrier()
        ocopies = [
            pltpu.async_copy(
                acc.at[pl.ds(row0 + k * CHUNK, CHUNK)],
                acc_out.at[pl.ds(c * N_PAD + row0 + k * CHUNK, CHUNK)],
                semg1)
            for k in range(ROWS_PER_SUB // CHUNK)
        ]
        for oc in ocopies:
            oc.wait()

    return pl.kernel(body, out_type=out_type, mesh=mesh,
                     scratch_types=scratch, compiler_params=_SC_PARAMS)


_agg = _make_agg()


def _make_cnt():
    mesh = plsc.VectorSubcoreMesh(core_axis_name="c", subcore_axis_name="s")
    out_type = [jax.ShapeDtypeStruct((NW * N_PAD,), jnp.float32),
                jax.ShapeDtypeStruct((NW, NCHUNK, CHUNK), jnp.int32)]
    scratch = [
        pltpu.VMEM((EPW,), jnp.int32),
        pltpu.VMEM((EPW,), jnp.int32),
        pltpu.VMEM((NCHUNK, CHUNK), jnp.int32),
        pltpu.VMEM((TAIL,), jnp.int32),
        pltpu.VMEM((N_PAD,), jnp.float32),
    ]

    def body(srcall_hbm, dstall_hbm, cnt_out, pk_out, src_v, dst_v,
             pk_v, tdst_v, cnt_v):
        c = lax.axis_index("c")
        s_ = lax.axis_index("s")
        w = s_ * NC + c
        off = pl.multiple_of(w * EPW, EPW)
        pltpu.sync_copy(srcall_hbm.at[pl.ds(off, EPW)], src_v)
        pltpu.sync_copy(dstall_hbm.at[pl.ds(off, EPW)], dst_v)
        toff = pl.multiple_of(E_MAIN + w * TAIL, TAIL)
        pltpu.sync_copy(dstall_hbm.at[pl.ds(toff, TAIL)], tdst_v)
        z16 = jnp.zeros((16,), jnp.float32)
        ones = jnp.ones((16,), jnp.float32)

        def zcnt_store(r, carry):
            cnt_v[pl.ds(r * 16, 16)] = z16
            return carry

        lax.fori_loop(0, N_PAD // 16, zcnt_store, 0)

        def step(i, carry):
            for j in range(CHUNK // 16):
                vs = src_v[pl.ds(i * CHUNK + j * 16, 16)]
                vd = dst_v[pl.ds(i * CHUNK + j * 16, 16)]
                pk_v[i, pl.ds(j * 16, 16)] = jnp.bitwise_or(
                    lax.shift_left(vs, 16), vd)
                plsc.addupdate_scatter(cnt_v, [vd], ones)
            return carry

        lax.fori_loop(0, NCHUNK, step, 0)
        plsc.addupdate_scatter(cnt_v, [tdst_v[...]], ones)
        pltpu.sync_copy(cnt_v, cnt_out.at[pl.ds(w * N_PAD, N_PAD)])
        pltpu.sync_copy(pk_v, pk_out.at[w])

    return pl.kernel(body, out_type=out_type, mesh=mesh,
                     scratch_types=scratch, compiler_params=_SC_PARAMS)


_cnt = _make_cnt()

BLK = 2048


def _combine_body(acc_ref, cnt_ref, x_ref, wl_ref, bl_ref, wr_ref, o_ref):
    cnt = jnp.sum(cnt_ref[...], axis=0)
    tot = jnp.sum(acc_ref[...], axis=0)
    mean = tot / jnp.maximum(cnt, 1.0)[:, None]
    h = lax.dot_general(mean, wl_ref[...], (((1,), (1,)), ((), ())),
                        preferred_element_type=jnp.float32)
    h = h + lax.dot_general(x_ref[...], wr_ref[...], (((1,), (1,)), ((), ())),
                            preferred_element_type=jnp.float32)
    h = h + bl_ref[...]
    o_ref[...] = jnp.maximum(h, 0.0)


def _combine(acc, cnt, x, wl, bl, wr):
    grid = (N_PAD // BLK,)
    return pl.pallas_call(
        _combine_body,
        grid=grid,
        in_specs=[
            pl.BlockSpec((NC, BLK, D), lambda i: (0, i, 0)),
            pl.BlockSpec((NW, BLK), lambda i: (0, i)),
            pl.BlockSpec((BLK, D), lambda i: (i, 0)),
            pl.BlockSpec((D, D), lambda i: (0, 0)),
            pl.BlockSpec((1, D), lambda i: (0, 0)),
            pl.BlockSpec((D, D), lambda i: (0, 0)),
        ],
        out_specs=pl.BlockSpec((BLK, D), lambda i: (i, 0)),
        out_shape=jax.ShapeDtypeStruct((N_NODES, D), jnp.float32),
    )(acc, cnt, x, wl, bl, wr)


@jax.jit
def kernel(x, edge_index, Wl1, bl1, Wr1, Wl2, bl2, Wr2):
    src = edge_index[0].astype(jnp.int32)
    dst = edge_index[1].astype(jnp.int32)
    bl1r = bl1.reshape(1, D)
    bl2r = bl2.reshape(1, D)

    cnt, pkm = _cnt(src, dst)
    cnt = cnt.reshape(NW, N_PAD)
    (acc1,) = _agg(pkm, x, src, dst)
    h = _combine(acc1.reshape(NC, N_PAD, D), cnt, x, Wl1, bl1r, Wr1)
    (acc2,) = _agg(pkm, h, src, dst)
    return _combine(acc2.reshape(NC, N_PAD, D), cnt, h, Wl2, bl2r, Wr2)
